# Initial kernel scaffold; baseline (speedup 1.0000x reference)
import jax
import jax.numpy as jnp
from jax.experimental import pallas as pl


def kernel(weights):
    raise NotImplementedError("write your pallas kernel here")


# speedup vs baseline: 67.2235x; 67.2235x over previous
import functools
import math

import jax
import jax.numpy as jnp
from jax import lax
from jax.experimental import pallas as pl
from jax.experimental.pallas import tpu as pltpu
from jax.experimental.pallas import tpu_sc as plsc

LANES = 16


def _allmax(v, lanes):
    for k in (1, 2, 4, 8):
        v = jnp.maximum(v, jnp.take(v, lanes ^ k))
    return v


def _allmin(v, lanes):
    for k in (1, 2, 4, 8):
        v = jnp.minimum(v, jnp.take(v, lanes ^ k))
    return v


@jax.jit
def kernel(weights):
    B, V, U = weights.shape
    assert U == 1024 and V % LANES == 0
    t0 = math.floor(V / math.e - 1.0) + 1
    n_eff = V - t0
    S = 48
    nb = 2 * ((n_eff + 2 * S - 1) // (2 * S))
    start = V - nb * S
    assert start >= 0 and start % 8 == 0
    nchunks = U // LANES
    nvldchunks = nchunks // 2
    ngroups = 4
    gsz = nchunks // ngroups

    info = plsc.get_sparse_core_info()
    nc, ns = info.num_cores, info.num_subcores
    assert B <= nc * ns

    mesh = plsc.VectorSubcoreMesh(core_axis_name="c", subcore_axis_name="s")

    @functools.partial(
        pl.kernel,
        mesh=mesh,
        out_type=(
            jax.ShapeDtypeStruct((B, LANES), jnp.float32),
            jax.ShapeDtypeStruct((B, V), jnp.int32),
        ),
        scratch_types=[
            pltpu.VMEM((S, U), jnp.float32),
            pltpu.VMEM((S, U), jnp.float32),
            pltpu.VMEM((U // 2,), jnp.uint32),
            pltpu.VMEM((2 * LANES,), jnp.int32),
            pltpu.VMEM((V,), jnp.int32),
            pltpu.VMEM((LANES,), jnp.float32),
            pltpu.SMEM((1,), jnp.int32),
            pltpu.SemaphoreType.DMA,
            pltpu.SemaphoreType.DMA,
        ],
    )
    def greedy_sc(w_hbm, size_hbm, seq_hbm, wb0, wb1, mref, bref, selsref,
                  ovec, cntref, sem0, sem1):
        wid = lax.axis_index("s") * nc + lax.axis_index("c")

        @pl.when(wid < B)
        def _worker():
            b = wid
            lanes = lax.iota(jnp.int32, LANES)
            one_i = jnp.int32(1)
            one_u = jnp.uint32(1)
            big_u = jnp.uint32(2**30)
            lanes_u = lanes.astype(jnp.uint32)
            z0u = jnp.uint32(0) - jnp.minimum(lanes_u, one_u)
            oh0 = (one_i - jnp.minimum(lanes, 1)).astype(jnp.float32)

            sent = jnp.full((LANES,), jnp.uint32(0xFFFFFFFF))
            for ci in range(nvldchunks):
                mref[pl.ds(ci * LANES, LANES)] = sent
            ones_i = jnp.full((LANES,), jnp.int32(-1))
            bref[pl.ds(0, LANES)] = ones_i
            bref[pl.ds(LANES, LANES)] = ones_i
            zi = jnp.zeros((LANES,), jnp.int32)
            for ci in range(V // LANES):
                selsref[pl.ds(ci * LANES, LANES)] = zi
            cntref[0] = jnp.int32(0)

            def issue(blk, wb, sem):
                return pltpu.async_copy(
                    w_hbm.at[b, pl.ds(start + blk * S, S), :], wb, sem)

            def wait(blk, wb, sem):
                pltpu.make_async_copy(
                    w_hbm.at[b, pl.ds(start + blk * S, S), :], wb, sem).wait()

            issue(0, wb0, sem0)

            def _scan_step(i, si, wb, mA, mB, acc):
                ks, cs = [], []
                for g in range(ngroups):
                    c0 = g * gsz
                    kg = None
                    cg = jnp.full((LANES,), jnp.uint32(c0 * LANES))
                    for ci in range(c0, c0 + gsz):
                        kv = lax.bitcast_convert_type(
                            wb[si, pl.ds(ci * LANES, LANES)], jnp.uint32)
                        if ci == 0:
                            kv = kv & z0u
                        if ci < nvldchunks:
                            kv = jnp.minimum(
                                kv, mref[pl.ds(ci * LANES, LANES)])
                        else:
                            p = ci - nvldchunks
                            m = mA if p < LANES else mB
                            am = lax.shift_right_arithmetic(
                                lax.shift_left(m, 31 - (p % LANES)), 31)
                            kv = kv & lax.bitcast_convert_type(
                                am, jnp.uint32)
                        if ci == c0:
                            kg = kv
                        else:
                            newmax = jnp.maximum(kg, kv)
                            ind = jnp.minimum(newmax - kg, one_u)
                            cg = jnp.maximum(cg, ind * (ci * LANES))
                            kg = newmax
                    ks.append(kg)
                    cs.append(cg)
                while len(ks) > 1:
                    nks, ncs = [], []
                    for j in range(0, len(ks), 2):
                        k1, c1, k2, c2 = ks[j], cs[j], ks[j+1], cs[j+1]
                        nk = jnp.maximum(k1, k2)
                        ind = jnp.minimum(nk - k1, one_u)
                        ncs.append(jnp.maximum(c1, ind * c2))
                        nks.append(nk)
                    ks, cs = nks, ncs
                kbest, cbest = ks[0], cs[0]
                kmax = _allmax(kbest, lanes)

                gidx = cbest + lanes_u
                sel = gidx + jnp.minimum(kmax - kbest, one_u) * big_u
                sel = _allmin(sel, lanes)
                sel_s = sel[0].astype(jnp.int32)
                mxv = lax.bitcast_convert_type(kmax, jnp.float32)

                g_take = one_i - lax.shift_right_logical(
                    jnp.int32(i - t0), 31)
                g_mark = g_take * jnp.minimum(sel_s, one_i)

                io = i & 15
                sbase = i - io
                d2 = lanes - io
                a2 = jnp.maximum(jnp.minimum(d2 * d2, one_i),
                                 one_i - g_take)
                s_old = selsref[pl.ds(sbase, LANES)]
                selsref[pl.ds(sbase, LANES)] = (
                    s_old * a2 + sel_s * (one_i - a2))

                cntref[0] = cntref[0] + g_mark

                mo = sel_s & 15
                d3 = lanes - mo
                oh3 = one_i - jnp.minimum(d3 * d3, one_i)

                g_hi = lax.shift_right_logical(
                    jnp.int32(sel_s - U // 2), 31)
                g_lo = g_mark * g_hi
                mbase = jnp.minimum(sel_s - mo, U // 2 - LANES)
                ohg = (oh3 * g_lo).astype(jnp.uint32)
                m_old = mref[pl.ds(mbase, LANES)]
                mref[pl.ds(mbase, LANES)] = m_old ^ (jnp.uint32(0) - ohg)

                jsel = sel_s >> 8
                p = (sel_s >> 4) & 15
                d0 = oh3 << p
                dA = jsel - 2
                gA = g_mark * (one_i - jnp.minimum(dA * dA, one_i))
                dB = jsel - 3
                gB = g_mark * (one_i - jnp.minimum(dB * dB, one_i))
                mA = mA ^ (d0 * gA)
                mB = mB ^ (d0 * gB)

                acc = acc + mxv[0] * g_take.astype(jnp.float32)
                return mA, mB, acc

            def do_block(blk, wb, acc):
                mA0 = bref[pl.ds(0, LANES)]
                mB0 = bref[pl.ds(LANES, LANES)]

                def step(si, carry):
                    mA, mB, a = carry
                    i = start + blk * S + si
                    return _scan_step(i, si, wb, mA, mB, a)

                mA, mB, acc = lax.fori_loop(0, S, step, (mA0, mB0, acc))
                bref[pl.ds(0, LANES)] = mA
                bref[pl.ds(LANES, LANES)] = mB
                return acc

            dead_blocks = (t0 - start) // S

            def outer(o, acc):
                for k in range(2):
                    blk = 2 * o + k
                    wb, sem = (wb0, sem0) if k == 0 else (wb1, sem1)
                    nwb, nsem = (wb1, sem1) if k == 0 else (wb0, sem0)

                    @pl.when(blk + 1 < nb)
                    def _prefetch():
                        issue(blk + 1, nwb, nsem)

                    wait(blk, wb, sem)
                    live = (blk >= dead_blocks) & (cntref[0] < U - 1)
                    acc = lax.cond(live,
                                   lambda a, _wb=wb, _blk=blk:
                                       do_block(_blk, _wb, a),
                                   lambda a: a, acc)
                return acc

            acc = lax.fori_loop(0, nb // 2, outer, jnp.float32(0.0))

            ovec[...] = oh0 * (-acc)
            pltpu.sync_copy(ovec, size_hbm.at[b])
            pltpu.sync_copy(selsref, seq_hbm.at[b])

    size_rows, seqs = greedy_sc(weights)
    return size_rows[:, 0], seqs


# speedup vs baseline: 67.3031x; 1.0012x over previous
import functools
import math

import jax
import jax.numpy as jnp
from jax import lax
from jax.experimental import pallas as pl
from jax.experimental.pallas import tpu as pltpu
from jax.experimental.pallas import tpu_sc as plsc

LANES = 16


def _allmax(v, lanes):
    for k in (1, 2, 4, 8):
        v = jnp.maximum(v, jnp.take(v, lanes ^ k))
    return v


def _allmin(v, lanes):
    for k in (1, 2, 4, 8):
        v = jnp.minimum(v, jnp.take(v, lanes ^ k))
    return v


@jax.jit
def kernel(weights):
    B, V, U = weights.shape
    assert U == 1024 and V % LANES == 0
    t0 = math.floor(V / math.e - 1.0) + 1
    n_eff = V - t0
    S = 48
    nb = 2 * ((n_eff + 2 * S - 1) // (2 * S))
    start = V - nb * S
    assert start >= 0 and start % 8 == 0
    nchunks = U // LANES
    nvldchunks = nchunks // 2
    ngroups = 4
    gsz = nchunks // ngroups

    info = plsc.get_sparse_core_info()
    nc, ns = info.num_cores, info.num_subcores
    assert B <= nc * ns

    mesh = plsc.VectorSubcoreMesh(core_axis_name="c", subcore_axis_name="s")

    @functools.partial(
        pl.kernel,
        mesh=mesh,
        out_type=(
            jax.ShapeDtypeStruct((B, LANES), jnp.float32),
            jax.ShapeDtypeStruct((B, V), jnp.int32),
        ),
        scratch_types=[
            pltpu.VMEM((S, U), jnp.float32),
            pltpu.VMEM((S, U), jnp.float32),
            pltpu.VMEM((U // 2,), jnp.uint32),
            pltpu.VMEM((2 * LANES,), jnp.int32),
            pltpu.VMEM((V,), jnp.int32),
            pltpu.VMEM((LANES,), jnp.float32),
            pltpu.SMEM((1,), jnp.int32),
            pltpu.SemaphoreType.DMA,
            pltpu.SemaphoreType.DMA,
        ],
    )
    def greedy_sc(w_hbm, size_hbm, seq_hbm, wb0, wb1, mref, bref, selsref,
                  ovec, cntref, sem0, sem1):
        wid = lax.axis_index("s") * nc + lax.axis_index("c")

        @pl.when(wid < B)
        def _worker():
            b = wid
            lanes = lax.iota(jnp.int32, LANES)
            one_i = jnp.int32(1)
            one_u = jnp.uint32(1)
            big_u = jnp.uint32(2**30)
            lanes_u = lanes.astype(jnp.uint32)
            z0u = jnp.uint32(0) - jnp.minimum(lanes_u, one_u)
            oh0 = (one_i - jnp.minimum(lanes, 1)).astype(jnp.float32)

            sent = jnp.full((LANES,), jnp.uint32(0xFFFFFFFF))
            for ci in range(nvldchunks):
                mref[pl.ds(ci * LANES, LANES)] = sent
            ones_i = jnp.full((LANES,), jnp.int32(-1))
            bref[pl.ds(0, LANES)] = ones_i
            bref[pl.ds(LANES, LANES)] = ones_i
            zi = jnp.zeros((LANES,), jnp.int32)
            for ci in range(V // LANES):
                selsref[pl.ds(ci * LANES, LANES)] = zi
            cntref[0] = jnp.int32(0)

            def issue(blk, wb, sem):
                return pltpu.async_copy(
                    w_hbm.at[b, pl.ds(start + blk * S, S), :], wb, sem)

            def wait(blk, wb, sem):
                pltpu.make_async_copy(
                    w_hbm.at[b, pl.ds(start + blk * S, S), :], wb, sem).wait()

            issue(0, wb0, sem0)

            def _scan_step(i, si, wb, mA, mB, acc):
                ks, cs = [], []
                for g in range(ngroups):
                    c0 = g * gsz
                    kg = None
                    cg = jnp.full((LANES,), jnp.uint32(c0 * LANES))
                    for ci in range(c0, c0 + gsz):
                        kv = lax.bitcast_convert_type(
                            wb[si, pl.ds(ci * LANES, LANES)], jnp.uint32)
                        if ci == 0:
                            kv = kv & z0u
                        if ci < nvldchunks:
                            kv = jnp.minimum(
                                kv, mref[pl.ds(ci * LANES, LANES)])
                        else:
                            p = ci - nvldchunks
                            m = mA if p < LANES else mB
                            am = lax.shift_right_arithmetic(
                                lax.shift_left(m, 31 - (p % LANES)), 31)
                            kv = kv & lax.bitcast_convert_type(
                                am, jnp.uint32)
                        if ci == c0:
                            kg = kv
                        else:
                            newmax = jnp.maximum(kg, kv)
                            ind = jnp.minimum(newmax - kg, one_u)
                            cg = jnp.maximum(cg, ind * (ci * LANES))
                            kg = newmax
                    ks.append(kg)
                    cs.append(cg)
                while len(ks) > 1:
                    nks, ncs = [], []
                    for j in range(0, len(ks), 2):
                        k1, c1, k2, c2 = ks[j], cs[j], ks[j+1], cs[j+1]
                        nk = jnp.maximum(k1, k2)
                        ind = jnp.minimum(nk - k1, one_u)
                        ncs.append(jnp.maximum(c1, ind * c2))
                        nks.append(nk)
                    ks, cs = nks, ncs
                kbest, cbest = ks[0], cs[0]
                kmax = _allmax(kbest, lanes)

                gidx = cbest + lanes_u
                sel = gidx + jnp.minimum(kmax - kbest, one_u) * big_u
                sel = _allmin(sel, lanes)
                sel_s = sel[0].astype(jnp.int32)
                mxv = lax.bitcast_convert_type(kmax, jnp.float32)

                g_take = one_i - lax.shift_right_logical(
                    jnp.int32(i - t0), 31)
                g_mark = g_take * jnp.minimum(sel_s, one_i)

                io = i & 15
                sbase = i - io
                d2 = lanes - io
                a2 = jnp.maximum(jnp.minimum(d2 * d2, one_i),
                                 one_i - g_take)
                s_old = selsref[pl.ds(sbase, LANES)]
                selsref[pl.ds(sbase, LANES)] = (
                    s_old * a2 + sel_s * (one_i - a2))

                cntref[0] = cntref[0] + g_mark

                mo = sel_s & 15
                d3 = lanes - mo
                oh3 = one_i - jnp.minimum(d3 * d3, one_i)

                g_hi = lax.shift_right_logical(
                    jnp.int32(sel_s - U // 2), 31)
                g_lo = g_mark * g_hi
                mbase = jnp.minimum(sel_s - mo, U // 2 - LANES)
                ohg = (oh3 * g_lo).astype(jnp.uint32)
                m_old = mref[pl.ds(mbase, LANES)]
                mref[pl.ds(mbase, LANES)] = m_old ^ (jnp.uint32(0) - ohg)

                jsel = sel_s >> 8
                p = (sel_s >> 4) & 15
                d0 = oh3 << p
                dA = jsel - 2
                gA = g_mark * (one_i - jnp.minimum(dA * dA, one_i))
                dB = jsel - 3
                gB = g_mark * (one_i - jnp.minimum(dB * dB, one_i))
                mA = mA ^ (d0 * gA)
                mB = mB ^ (d0 * gB)

                acc = acc + mxv[0] * g_take.astype(jnp.float32)
                return mA, mB, acc

            def do_block(blk, wb, acc):
                mA0 = bref[pl.ds(0, LANES)]
                mB0 = bref[pl.ds(LANES, LANES)]

                def step(so, carry):
                    mA, mB, a = carry
                    si = so * 2
                    i = start + blk * S + si
                    mA, mB, a = _scan_step(i, si, wb, mA, mB, a)
                    return _scan_step(i + 1, si + 1, wb, mA, mB, a)

                assert S % 2 == 0
                mA, mB, acc = lax.fori_loop(0, S // 2, step,
                                            (mA0, mB0, acc))
                bref[pl.ds(0, LANES)] = mA
                bref[pl.ds(LANES, LANES)] = mB
                return acc

            dead_blocks = (t0 - start) // S

            def outer(o, acc):
                for k in range(2):
                    blk = 2 * o + k
                    wb, sem = (wb0, sem0) if k == 0 else (wb1, sem1)
                    nwb, nsem = (wb1, sem1) if k == 0 else (wb0, sem0)

                    @pl.when(blk + 1 < nb)
                    def _prefetch():
                        issue(blk + 1, nwb, nsem)

                    wait(blk, wb, sem)
                    live = (blk >= dead_blocks) & (cntref[0] < U - 1)
                    acc = lax.cond(live,
                                   lambda a, _wb=wb, _blk=blk:
                                       do_block(_blk, _wb, a),
                                   lambda a: a, acc)
                return acc

            acc = lax.fori_loop(0, nb // 2, outer, jnp.float32(0.0))

            ovec[...] = oh0 * (-acc)
            pltpu.sync_copy(ovec, size_hbm.at[b])
            pltpu.sync_copy(selsref, seq_hbm.at[b])

    size_rows, seqs = greedy_sc(weights)
    return size_rows[:, 0], seqs


# speedup vs baseline: 67.4187x; 1.0017x over previous
import functools
import math

import jax
import jax.numpy as jnp
from jax import lax
from jax.experimental import pallas as pl
from jax.experimental.pallas import tpu as pltpu
from jax.experimental.pallas import tpu_sc as plsc

LANES = 16


def _allmax(v, lanes):
    for k in (1, 2, 4, 8):
        v = jnp.maximum(v, jnp.take(v, lanes ^ k))
    return v


def _allmin(v, lanes):
    for k in (1, 2, 4, 8):
        v = jnp.minimum(v, jnp.take(v, lanes ^ k))
    return v


@jax.jit
def kernel(weights):
    B, V, U = weights.shape
    assert U == 1024 and V % LANES == 0
    t0 = math.floor(V / math.e - 1.0) + 1
    n_eff = V - t0
    S = 48
    nb = 2 * ((n_eff + 2 * S - 1) // (2 * S))
    start = V - nb * S
    assert start >= 0 and start % 8 == 0
    nchunks = U // LANES
    nvldchunks = nchunks // 2
    ngroups = 4
    gsz = nchunks // ngroups

    info = plsc.get_sparse_core_info()
    nc, ns = info.num_cores, info.num_subcores
    assert B <= nc * ns

    mesh = plsc.VectorSubcoreMesh(core_axis_name="c", subcore_axis_name="s")

    @functools.partial(
        pl.kernel,
        mesh=mesh,
        out_type=(
            jax.ShapeDtypeStruct((B, LANES), jnp.float32),
            jax.ShapeDtypeStruct((B, V), jnp.int32),
        ),
        scratch_types=[
            pltpu.VMEM((S, U), jnp.float32),
            pltpu.VMEM((S, U), jnp.float32),
            pltpu.VMEM((U // 2,), jnp.uint32),
            pltpu.VMEM((2 * LANES,), jnp.int32),
            pltpu.VMEM((V,), jnp.int32),
            pltpu.VMEM((LANES,), jnp.float32),
            pltpu.SMEM((1,), jnp.int32),
            pltpu.SemaphoreType.DMA,
            pltpu.SemaphoreType.DMA,
        ],
    )
    def greedy_sc(w_hbm, size_hbm, seq_hbm, wb0, wb1, mref, bref, selsref,
                  ovec, cntref, sem0, sem1):
        wid = lax.axis_index("s") * nc + lax.axis_index("c")

        @pl.when(wid < B)
        def _worker():
            b = wid
            lanes = lax.iota(jnp.int32, LANES)
            one_i = jnp.int32(1)
            one_u = jnp.uint32(1)
            big_u = jnp.uint32(2**30)
            lanes_u = lanes.astype(jnp.uint32)
            z0u = jnp.uint32(0) - jnp.minimum(lanes_u, one_u)
            oh0 = (one_i - jnp.minimum(lanes, 1)).astype(jnp.float32)

            sent = jnp.full((LANES,), jnp.uint32(0xFFFFFFFF))
            for ci in range(nvldchunks):
                mref[pl.ds(ci * LANES, LANES)] = sent
            ones_i = jnp.full((LANES,), jnp.int32(-1))
            bref[pl.ds(0, LANES)] = ones_i
            bref[pl.ds(LANES, LANES)] = ones_i
            zi = jnp.zeros((LANES,), jnp.int32)
            for ci in range(V // LANES):
                selsref[pl.ds(ci * LANES, LANES)] = zi
            cntref[0] = jnp.int32(0)

            def issue(blk, wb, sem):
                return pltpu.async_copy(
                    w_hbm.at[b, pl.ds(start + blk * S, S), :], wb, sem)

            def wait(blk, wb, sem):
                pltpu.make_async_copy(
                    w_hbm.at[b, pl.ds(start + blk * S, S), :], wb, sem).wait()

            issue(0, wb0, sem0)

            def _scan_step(i, si, wb, mA, mB, acc):
                ks, cs = [], []
                for g in range(ngroups):
                    c0 = g * gsz
                    kg = None
                    cg = jnp.full((LANES,), jnp.uint32(c0 * LANES))
                    for ci in range(c0, c0 + gsz):
                        kv = lax.bitcast_convert_type(
                            wb[si, pl.ds(ci * LANES, LANES)], jnp.uint32)
                        if ci == 0:
                            kv = kv & z0u
                        if ci < nvldchunks:
                            kv = jnp.minimum(
                                kv, mref[pl.ds(ci * LANES, LANES)])
                        else:
                            p = ci - nvldchunks
                            m = mA if p < LANES else mB
                            am = lax.shift_right_arithmetic(
                                lax.shift_left(m, 31 - (p % LANES)), 31)
                            kv = kv & lax.bitcast_convert_type(
                                am, jnp.uint32)
                        if ci == c0:
                            kg = kv
                        else:
                            newmax = jnp.maximum(kg, kv)
                            ind = jnp.minimum(newmax - kg, one_u)
                            cg = jnp.maximum(cg, ind * (ci * LANES))
                            kg = newmax
                    ks.append(kg)
                    cs.append(cg)
                while len(ks) > 1:
                    nks, ncs = [], []
                    for j in range(0, len(ks), 2):
                        k1, c1, k2, c2 = ks[j], cs[j], ks[j+1], cs[j+1]
                        nk = jnp.maximum(k1, k2)
                        ind = jnp.minimum(nk - k1, one_u)
                        ncs.append(jnp.maximum(c1, ind * c2))
                        nks.append(nk)
                    ks, cs = nks, ncs
                kbest, cbest = ks[0], cs[0]
                kmax = _allmax(kbest, lanes)

                gidx = cbest + lanes_u
                sel = gidx + jnp.minimum(kmax - kbest, one_u) * big_u
                sel = _allmin(sel, lanes)
                sel_s = sel[0].astype(jnp.int32)
                mxv = lax.bitcast_convert_type(kmax, jnp.float32)

                g_take = one_i - lax.shift_right_logical(
                    jnp.int32(i - t0), 31)
                g_mark = g_take * jnp.minimum(sel_s, one_i)

                io = i & 15
                sbase = i - io
                d2 = lanes - io
                a2 = jnp.maximum(jnp.minimum(d2 * d2, one_i),
                                 one_i - g_take)
                s_old = selsref[pl.ds(sbase, LANES)]
                selsref[pl.ds(sbase, LANES)] = (
                    s_old * a2 + sel_s * (one_i - a2))

                cntref[0] = cntref[0] + g_mark

                mo = sel_s & 15
                d3 = lanes - mo
                oh3 = one_i - jnp.minimum(d3 * d3, one_i)

                g_hi = lax.shift_right_logical(
                    jnp.int32(sel_s - U // 2), 31)
                g_lo = g_mark * g_hi
                mbase = jnp.minimum(sel_s - mo, U // 2 - LANES)
                ohg = (oh3 * g_lo).astype(jnp.uint32)
                m_old = mref[pl.ds(mbase, LANES)]
                mref[pl.ds(mbase, LANES)] = m_old ^ (jnp.uint32(0) - ohg)

                jsel = sel_s >> 8
                p = (sel_s >> 4) & 15
                d0 = oh3 << p
                dA = jsel - 2
                gA = g_mark * (one_i - jnp.minimum(dA * dA, one_i))
                dB = jsel - 3
                gB = g_mark * (one_i - jnp.minimum(dB * dB, one_i))
                mA = mA ^ (d0 * gA)
                mB = mB ^ (d0 * gB)

                acc = acc + mxv[0] * g_take.astype(jnp.float32)
                return mA, mB, acc

            def do_block(blk, wb, acc):
                mA0 = bref[pl.ds(0, LANES)]
                mB0 = bref[pl.ds(LANES, LANES)]

                def step(si, carry):
                    mA, mB, a = carry
                    i = start + blk * S + si
                    return _scan_step(i, si, wb, mA, mB, a)

                mA, mB, acc = lax.fori_loop(0, S, step, (mA0, mB0, acc))
                bref[pl.ds(0, LANES)] = mA
                bref[pl.ds(LANES, LANES)] = mB
                return acc

            dead_blocks = (t0 - start) // S

            def outer(o, acc):
                for k in range(2):
                    blk = 2 * o + k
                    wb, sem = (wb0, sem0) if k == 0 else (wb1, sem1)
                    nwb, nsem = (wb1, sem1) if k == 0 else (wb0, sem0)

                    @pl.when(blk + 1 < nb)
                    def _prefetch():
                        issue(blk + 1, nwb, nsem)

                    wait(blk, wb, sem)
                    live = (blk >= dead_blocks) & (cntref[0] < U - 1)
                    acc = lax.cond(live,
                                   lambda a, _wb=wb, _blk=blk:
                                       do_block(_blk, _wb, a),
                                   lambda a: a, acc)
                return acc

            acc = lax.fori_loop(0, nb // 2, outer, jnp.float32(0.0))

            ovec[...] = oh0 * (-acc)
            pltpu.sync_copy(ovec, size_hbm.at[b])
            pltpu.sync_copy(selsref, seq_hbm.at[b])

    size_rows, seqs = greedy_sc(weights)
    return size_rows[:, 0], seqs
